# Initial kernel scaffold; baseline (speedup 1.0000x reference)
import jax
import jax.numpy as jnp
from jax.experimental import pallas as pl


def kernel(class_logits, box_regression, proposals):
    raise NotImplementedError("write your pallas kernel here")


# speedup vs baseline: 20.2203x; 20.2203x over previous
import functools
import jax
import jax.numpy as jnp
from jax import lax
from jax.experimental import pallas as pl
from jax.experimental.pallas import tpu as pltpu
import numpy as np

_N = 10000
_NPAD = 10240
_ROWS = 80
_CROWS = 320
_C = 5
_SCORE_THRESH = 0.05
_NMS_THRESH = 0.5
_DETS = 100
_IMG = 512.0
_CLIP = float(np.log(1000.0 / 16.0))


def _nms_body(cl, br, pr, obox, oscr, olab,
              s, x1n, y1n, x2n, y2n, ar, x1o, y1o, x2o, y2o):
    w = pr[2] - pr[0]
    h = pr[3] - pr[1]
    cx = pr[0] + 0.5 * w
    cy = pr[1] + 0.5 * h

    l0, l1, l2, l3, l4 = cl[0], cl[1], cl[2], cl[3], cl[4]
    mx = jnp.maximum(jnp.maximum(jnp.maximum(l0, l1), jnp.maximum(l2, l3)), l4)
    e0 = jnp.exp(l0 - mx)
    e1 = jnp.exp(l1 - mx)
    e2 = jnp.exp(l2 - mx)
    e3 = jnp.exp(l3 - mx)
    e4 = jnp.exp(l4 - mx)
    den = e0 + e1 + e2 + e3 + e4
    es = [e1, e2, e3, e4]

    rr = lax.broadcasted_iota(jnp.int32, (_ROWS, 128), 0)
    cc = lax.broadcasted_iota(jnp.int32, (_ROWS, 128), 1)
    n_lin = rr * 128 + cc
    real = n_lin < _N

    mc = jnp.float32(0.0)
    for c in range(1, 5):
        dx = br[4 * c + 0] / 10.0
        dy = br[4 * c + 1] / 10.0
        dw = jnp.minimum(br[4 * c + 2] / 5.0, _CLIP)
        dh = jnp.minimum(br[4 * c + 3] / 5.0, _CLIP)
        px = dx * w + cx
        py = dy * h + cy
        pw = jnp.exp(dw) * w
        ph = jnp.exp(dh) * h
        x1 = jnp.clip(px - 0.5 * pw, 0.0, _IMG)
        x2 = jnp.clip(px + 0.5 * pw, 0.0, _IMG)
        y1 = jnp.clip(py - 0.5 * ph, 0.0, _IMG)
        y2 = jnp.clip(py + 0.5 * ph, 0.0, _IMG)
        sc = es[c - 1] / den
        keep = (sc > _SCORE_THRESH) & ((x2 - x1) >= 0.01) & ((y2 - y1) >= 0.01) & real
        ms = jnp.where(keep, sc, -1.0)
        r0 = (c - 1) * _ROWS
        x1o[r0:r0 + _ROWS] = x1
        y1o[r0:r0 + _ROWS] = y1
        x2o[r0:r0 + _ROWS] = x2
        y2o[r0:r0 + _ROWS] = y2
        s[r0:r0 + _ROWS] = ms
        m4 = jnp.maximum(jnp.maximum(jnp.max(x1), jnp.max(x2)),
                         jnp.maximum(jnp.max(y1), jnp.max(y2)))
        mc = jnp.maximum(mc, m4)

    off_base = mc + 1.0
    for c in range(1, 5):
        r0 = (c - 1) * _ROWS
        off = jnp.float32(c) * off_base
        a1 = x1o[r0:r0 + _ROWS] + off
        b1 = y1o[r0:r0 + _ROWS] + off
        a2 = x2o[r0:r0 + _ROWS] + off
        b2 = y2o[r0:r0 + _ROWS] + off
        x1n[r0:r0 + _ROWS] = a1
        y1n[r0:r0 + _ROWS] = b1
        x2n[r0:r0 + _ROWS] = a2
        y2n[r0:r0 + _ROWS] = b2
        ar[r0:r0 + _ROWS] = (a2 - a1) * (b2 - b1)

    lr = lax.broadcasted_iota(jnp.int32, (_CROWS, 128), 0)
    lc = lax.broadcasted_iota(jnp.int32, (_CROWS, 128), 1)
    lin = lr * 128 + lc
    lane = lax.iota(jnp.int32, 128)

    def it(i, carry):
        vb1, vb2, vb3, vb4, vsc, vlab = carry
        sv = s[:]
        m = jnp.max(sv)
        sel = sv == m
        idx = jnp.min(jnp.where(sel, lin, jnp.int32(2 ** 30)))
        valid = m > 0.0
        smask = lin == idx
        bo1 = jnp.sum(jnp.where(smask, x1o[:], 0.0))
        bo2 = jnp.sum(jnp.where(smask, y1o[:], 0.0))
        bo3 = jnp.sum(jnp.where(smask, x2o[:], 0.0))
        bo4 = jnp.sum(jnp.where(smask, y2o[:], 0.0))
        labi = idx // _NPAD + 1
        off = labi.astype(jnp.float32) * off_base
        bx1 = bo1 + off
        by1 = bo2 + off
        bx2 = bo3 + off
        by2 = bo4 + off
        area1 = (bx2 - bx1) * (by2 - by1)
        ltx = jnp.maximum(bx1, x1n[:])
        lty = jnp.maximum(by1, y1n[:])
        rbx = jnp.minimum(bx2, x2n[:])
        rby = jnp.minimum(by2, y2n[:])
        inter = jnp.maximum(rbx - ltx, 0.0) * jnp.maximum(rby - lty, 0.0)
        iou = inter / (area1 + ar[:] - inter + 1e-9)
        sup = (iou > _NMS_THRESH) | smask
        s[:] = jnp.where(sup, -1.0, sv)
        wl = lane == i
        vf = jnp.where(valid, 1.0, 0.0)
        vb1 = jnp.where(wl, bo1 * vf, vb1)
        vb2 = jnp.where(wl, bo2 * vf, vb2)
        vb3 = jnp.where(wl, bo3 * vf, vb3)
        vb4 = jnp.where(wl, bo4 * vf, vb4)
        vsc = jnp.where(wl, m * vf, vsc)
        vlab = jnp.where(wl, jnp.where(valid, labi, 0), vlab)
        return vb1, vb2, vb3, vb4, vsc, vlab

    z = jnp.zeros((128,), jnp.float32)
    zi = jnp.zeros((128,), jnp.int32)
    vb1, vb2, vb3, vb4, vsc, vlab = lax.fori_loop(
        0, _DETS, it, (z, z, z, z, z, zi))
    obox[0] = vb1
    obox[1] = vb2
    obox[2] = vb3
    obox[3] = vb4
    oscr[:] = vsc
    olab[:] = vlab


@jax.jit
def kernel(class_logits, box_regression, proposals):
    padn = _NPAD - _N
    clp = jnp.pad(class_logits, ((0, padn), (0, 0)))
    brp = jnp.pad(box_regression, ((0, padn), (0, 0)))
    prp = jnp.pad(proposals, ((0, padn), (0, 0)))
    clT = clp.T.reshape(_C, _ROWS, 128)
    brT = brp.T.reshape(4 * _C, _ROWS, 128)
    prT = prp.T.reshape(4, _ROWS, 128)

    obox, oscr, olab = pl.pallas_call(
        _nms_body,
        out_shape=(
            jax.ShapeDtypeStruct((4, 128), jnp.float32),
            jax.ShapeDtypeStruct((128,), jnp.float32),
            jax.ShapeDtypeStruct((128,), jnp.int32),
        ),
        scratch_shapes=[pltpu.VMEM((_CROWS, 128), jnp.float32)
                        for _ in range(10)],
    )(clT, brT, prT)

    out_boxes = obox.T[:_DETS]
    out_scores = oscr[:_DETS]
    out_labels = olab[:_DETS]
    return out_boxes, out_scores, out_labels
